# baseline (speedup 1.0000x reference)
import jax
import jax.numpy as jnp
from jax.experimental import pallas as pl
from jax.experimental.pallas import tpu as pltpu

_N = 16384
_D = 1024
_BLOCK_ROWS = 3904


def _copy_body(x_ref, o_ref):
    o_ref[...] = x_ref[...]


def kernel(input):
    return pl.pallas_call(
        _copy_body,
        grid=(pl.cdiv(_N, _BLOCK_ROWS),),
        in_specs=[pl.BlockSpec((_BLOCK_ROWS, _D), lambda i: (i, 0))],
        out_specs=pl.BlockSpec((_BLOCK_ROWS, _D), lambda i: (i, 0)),
        out_shape=jax.ShapeDtypeStruct((_N, _D), jnp.float32),
        compiler_params=pltpu.CompilerParams(
            dimension_semantics=("parallel",),
            vmem_limit_bytes=67043328,
        ),
    )(input)


# speedup vs baseline: 1.0038x; 1.0038x over previous
import jax
import jax.numpy as jnp
from jax.experimental import pallas as pl
from jax.experimental.pallas import tpu as pltpu

_N = 16384
_D = 1024
_BLOCK_ROWS = 3776


def _copy_body(x_ref, o_ref):
    o_ref[...] = x_ref[...]


def kernel(input):
    return pl.pallas_call(
        _copy_body,
        grid=(pl.cdiv(_N, _BLOCK_ROWS),),
        in_specs=[pl.BlockSpec((_BLOCK_ROWS, _D), lambda i: (i, 0))],
        out_specs=pl.BlockSpec((_BLOCK_ROWS, _D), lambda i: (i, 0)),
        out_shape=jax.ShapeDtypeStruct((_N, _D), jnp.float32),
        compiler_params=pltpu.CompilerParams(
            dimension_semantics=("parallel",),
            vmem_limit_bytes=67043328,
        ),
    )(input)
